# baseline (speedup 1.0000x reference)
import functools

import jax
import jax.numpy as jnp
from jax import lax
from jax.experimental import pallas as pl
from jax.experimental.pallas import tpu as pltpu
from jax.experimental.pallas import tpu_sc as plsc

N = 10000
E = 320000
H = 128

NC = 2
NS = 16
NW = NC * NS

NP = 10240
EP = 327680
PER_W = EP // NW
C = 128
NCHUNK = PER_W // C
ROWS_PER_SUB = NP // NS

def _sc_mesh():
    return plsc.VectorSubcoreMesh(core_axis_name="c", subcore_axis_name="s",
                                  num_cores=NC, num_subcores=NS)


def _sc_gather2(table, rowp, colp, d):
    kk = 80
    tot = (EP // C) // NS
    kmax = max(kk, tot - kk)

    def body(tab, rid, cid, out_r, out_c, idxs, b00, b01, b10, b11,
             g00, g01, g10, g11, w00, w01, w10, w11):
        cor = lax.axis_index("c")
        sid = lax.axis_index("s")

        bufs = ((b00, b01), (b10, b11))
        gsem = ((g00, g01), (g10, g11))
        wsem = ((w00, w01), (w10, w11))
        outs = (out_r, out_c)

        def run(nch, base):
            pltpu.sync_copy(rid.at[pl.ds(base, nch * C)],
                            idxs.at[0, pl.ds(0, nch * C)])
            pltpu.sync_copy(cid.at[pl.ds(base, nch * C)],
                            idxs.at[1, pl.ds(0, nch * C)])

            def fire(b, p, g):
                pltpu.async_copy(
                    tab.at[idxs.at[b, pl.ds(g * C, C)]], bufs[b][p],
                    gsem[b][p])

            def wait_gather(b, p):
                pltpu.make_async_copy(
                    tab.at[pl.ds(0, C)], bufs[b][p], gsem[b][p]).wait()

            def wait_wb(b, p):
                pltpu.make_async_copy(
                    bufs[b][p], outs[b].at[pl.ds(base, C)], wsem[b][p]).wait()

            for b in (0, 1):
                fire(b, 0, 0)

            @pl.loop(0, nch, step=2)
            def _(g0):
                for p in (0, 1):
                    g = g0 + p
                    for b in (0, 1):
                        wait_gather(b, p)
                        pltpu.async_copy(
                            bufs[b][p], outs[b].at[pl.ds(base + g * C, C)],
                            wsem[b][p])

                        @pl.when(g + 1 < nch)
                        def _():
                            @pl.when(g >= 1)
                            def _():
                                wait_wb(b, 1 - p)
                            fire(b, 1 - p, g + 1)

            for b in (0, 1):
                wait_wb(b, (nch - 1) % 2)

        @pl.when(cor == 0)
        def _():
            run(kk, sid * (kk * C))

        @pl.when(cor == 1)
        def _():
            run(tot - kk, (NS * kk + sid * (tot - kk)) * C)

    fn = pl.kernel(
        body,
        out_type=[
            jax.ShapeDtypeStruct((EP, d), jnp.float32),
            jax.ShapeDtypeStruct((EP, d), jnp.float32),
        ],
        mesh=_sc_mesh(),
        scratch_types=[
            pltpu.VMEM((2, kmax * C), jnp.int32),
            pltpu.VMEM((C, d), jnp.float32),
            pltpu.VMEM((C, d), jnp.float32),
            pltpu.VMEM((C, d), jnp.float32),
            pltpu.VMEM((C, d), jnp.float32),
            pltpu.SemaphoreType.DMA,
            pltpu.SemaphoreType.DMA,
            pltpu.SemaphoreType.DMA,
            pltpu.SemaphoreType.DMA,
            pltpu.SemaphoreType.DMA,
            pltpu.SemaphoreType.DMA,
            pltpu.SemaphoreType.DMA,
            pltpu.SemaphoreType.DMA,
        ],
        name=f"sc_gather2_{d}",
        compiler_params=pltpu.CompilerParams(use_tc_tiling_on_sc=(d % 128 == 0)),
    )
    return fn(table, rowp, colp)


def _sc_scatter_add(msg, colp, zeros_np):
    def body(m, cidx, zer, out0, out1, idx, buf, acc, sem):
        cid = lax.axis_index("c")
        sid = lax.axis_index("s")
        myrows = pl.ds(sid * ROWS_PER_SUB, ROWS_PER_SUB)
        pltpu.sync_copy(zer.at[myrows], acc.at[myrows])
        plsc.subcore_barrier()

        base = (sid * NC + cid) * PER_W

        @pl.loop(0, NCHUNK)
        def _(k):
            off = base + k * C
            pltpu.sync_copy(cidx.at[pl.ds(off, C)], idx)
            cp = pltpu.async_copy(m.at[pl.ds(off, C)], buf, sem)
            cp.wait()
            pltpu.sync_copy(buf, acc.at[idx], add=True)

        plsc.subcore_barrier()

        @pl.when(cid == 0)
        def _():
            pltpu.sync_copy(acc.at[myrows], out0.at[myrows])

        @pl.when(cid == 1)
        def _():
            pltpu.sync_copy(acc.at[myrows], out1.at[myrows])

    fn = pl.kernel(
        body,
        out_type=[
            jax.ShapeDtypeStruct((NP, H), jnp.float32),
            jax.ShapeDtypeStruct((NP, H), jnp.float32),
        ],
        mesh=_sc_mesh(),
        scratch_types=[
            pltpu.VMEM((C,), jnp.int32),
            pltpu.VMEM((C, H), jnp.float32),
            pltpu.VMEM_SHARED((NP, H), jnp.float32),
            pltpu.SemaphoreType.DMA,
        ],
        name="sc_scatter_add",
    )
    return fn(msg, colp, zeros_np)


def _ln(y, g, be):
    mu = jnp.mean(y, axis=-1, keepdims=True)
    var = jnp.mean((y - mu) * (y - mu), axis=-1, keepdims=True)
    return (y - mu) * lax.rsqrt(var + 1e-5) * g + be


def _full(shape):
    return pl.BlockSpec(shape, lambda i: (0,) * len(shape))


def _rows(b, d):
    return pl.BlockSpec((b, d), lambda i: (i, 0))


def _encode_body(x_ref, w1, b1, w2, b2, g, be, o_ref):
    h = jnp.maximum(x_ref[...] @ w1[...] + b1[...], 0.0)
    o_ref[...] = _ln(h @ w2[...] + b2[...], g[...], be[...])


def _tc_encode(x, w1, b1, w2, b2, g, be, bn):
    n, din = x.shape
    return pl.pallas_call(
        _encode_body,
        grid=(n // bn,),
        in_specs=[
            _rows(bn, din),
            _full(w1.shape), _full(b1.shape), _full(w2.shape),
            _full(b2.shape), _full(g.shape), _full(be.shape),
        ],
        out_specs=_rows(bn, H),
        out_shape=jax.ShapeDtypeStruct((n, H), jnp.float32),
        name="tc_encode",
    )(x, w1, b1, w2, b2, g, be)


def _edge_enc_body(pr_ref, pc_ref, ssel, wd, wdist, b1, w2, b2, g, be, o_ref):
    d = pr_ref[...] - pc_ref[...]
    s2 = (d * d) @ ssel[...]
    dist = jnp.sqrt(s2 + 1e-12)
    h = jnp.maximum(d @ wd[...] + dist @ wdist[...] + b1[...], 0.0)
    o_ref[...] = _ln(h @ w2[...] + b2[...], g[...], be[...])


def _tc_edge_encode(pr, pc, ssel, wd, wdist, b1, w2, b2, g, be, bn):
    return pl.pallas_call(
        _edge_enc_body,
        grid=(EP // bn,),
        in_specs=[
            _rows(bn, pr.shape[1]), _rows(bn, pc.shape[1]),
            _full(ssel.shape), _full(wd.shape), _full(wdist.shape),
            _full(b1.shape), _full(w2.shape), _full(b2.shape),
            _full(g.shape), _full(be.shape),
        ],
        out_specs=_rows(bn, H),
        out_shape=jax.ShapeDtypeStruct((EP, H), jnp.float32),
        name="tc_edge_encode",
    )(pr, pc, ssel, wd, wdist, b1, w2, b2, g, be)


def _dot(a, b):
    return jnp.dot(a, b, preferred_element_type=jnp.float32)


def _edge_step_body(xr_ref, xc_ref, eh_ref, w1ab, w1c, b1, w2, b2, g, be,
                    msg_ref, ne_ref):
    eh = eh_ref[...]
    bf = jnp.bfloat16
    mc = _dot(xc_ref[...], w1ab[...])
    mr = _dot(xr_ref[...], w1ab[...])
    ec = _dot(eh.astype(bf), w1c[...]) + b1[...]
    pre_m = mc[:, :H] + mr[:, H:] + ec
    pre_n = mr[:, :H] + mc[:, H:] + ec
    hm = jnp.maximum(pre_m, 0.0).astype(bf)
    hn = jnp.maximum(pre_n, 0.0).astype(bf)
    msg_ref[...] = _ln(_dot(hm, w2[...]) + b2[...], g[...], be[...])
    ne_ref[...] = eh + _ln(_dot(hn, w2[...]) + b2[...], g[...], be[...])


def _tc_edge_step(xr, xc, eh, w1ab, w1c, b1, w2, b2, g, be, bn):
    return pl.pallas_call(
        _edge_step_body,
        grid=(EP // bn,),
        in_specs=[
            _rows(bn, H), _rows(bn, H), _rows(bn, H),
            _full(w1ab.shape), _full(w1c.shape), _full(b1.shape),
            _full(w2.shape), _full(b2.shape), _full(g.shape), _full(be.shape),
        ],
        out_specs=[_rows(bn, H), _rows(bn, H)],
        out_shape=[
            jax.ShapeDtypeStruct((EP, H), jnp.float32),
            jax.ShapeDtypeStruct((EP, H), jnp.float32),
        ],
        name="tc_edge_step",
    )(xr, xc, eh, w1ab, w1c, b1, w2, b2, g, be)


def _node_step_body(a0_ref, a1_ref, xh_ref, wna, wnb, b1, w2, b2, g, be,
                    o_ref):
    xh = xh_ref[...]
    pre = (a0_ref[...] + a1_ref[...]) @ wna[...] + xh @ wnb[...] + b1[...]
    h = jnp.maximum(pre, 0.0)
    o_ref[...] = xh + _ln(h @ w2[...] + b2[...], g[...], be[...])


def _tc_node_step(a0, a1, xh, wna, wnb, b1, w2, b2, g, be, bn):
    return pl.pallas_call(
        _node_step_body,
        grid=(NP // bn,),
        in_specs=[
            _rows(bn, H), _rows(bn, H), _rows(bn, H),
            _full(wna.shape), _full(wnb.shape), _full(b1.shape),
            _full(w2.shape), _full(b2.shape), _full(g.shape), _full(be.shape),
        ],
        out_specs=_rows(bn, H),
        out_shape=jax.ShapeDtypeStruct((NP, H), jnp.float32),
        name="tc_node_step",
    )(a0, a1, xh, wna, wnb, b1, w2, b2, g, be)


def _decode_body(xh_ref, w1, b1, w2, b2, o_ref):
    h = jnp.maximum(xh_ref[...] @ w1[...] + b1[...], 0.0)
    o_ref[...] = h @ w2[...] + b2[...]


def _tc_decode(xh, w1, b1, w2, b2, bn):
    return pl.pallas_call(
        _decode_body,
        grid=(NP // bn,),
        in_specs=[
            _rows(bn, H),
            _full(w1.shape), _full(b1.shape), _full(w2.shape), _full(b2.shape),
        ],
        out_specs=_rows(bn, H),
        out_shape=jax.ShapeDtypeStruct((NP, H), jnp.float32),
        name="tc_decode",
    )(xh, w1, b1, w2, b2)


def _fold_norm(w1, b1, mu, sig):
    w1f = w1 / sig[:, None]
    b1f = b1 - (mu / sig) @ w1
    return w1f, b1f


def _row(v):
    return v.reshape(1, -1)


def kernel(world_pos, mesh_pos, phi, swelling_phi, swelling_phi_rate,
           node_type, time, mat_param, edge_index, params):
    f32 = jnp.float32

    u = world_pos - mesh_pos
    freqs = 2.0 ** jnp.arange(2, dtype=f32)
    temb = jnp.concatenate([jnp.sin(freqs * time), jnp.cos(freqs * time)])
    x = jnp.concatenate(
        [u, phi, swelling_phi, swelling_phi_rate, node_type,
         jnp.tile(temb[None, :], (N, 1)), jnp.tile(mat_param[None, :], (N, 1))],
        axis=-1)
    x = jnp.pad(x, ((0, NP - N), (0, 0)))

    ptab = jnp.concatenate(
        [mesh_pos, world_pos, phi, jnp.zeros((N, 11), f32)], axis=-1)
    ptab = jnp.pad(ptab, ((0, NP - N), (0, 0)))

    rowp = jnp.full((EP,), N, jnp.int32).at[:E].set(edge_index[0])
    colp = jnp.full((EP,), N, jnp.int32).at[:E].set(edge_index[1])

    nmu, nsig = params["node_norm"]
    (wn1, bn1), (wn2, bn2) = params["node_encoder"]["layers"]
    gn, ben = params["node_encoder"]["ln"]
    wn1f, bn1f = _fold_norm(wn1, bn1, nmu, nsig)

    emu, esig = params["edge_norm"]
    (we1, be1), (we2, be2) = params["edge_encoder"]["layers"]
    ge, bee = params["edge_encoder"]["ln"]
    we1f, be1f = _fold_norm(we1, be1, emu, esig)
    wd = jnp.zeros((16, H), f32)
    wd = wd.at[0].set(we1f[0]).at[1].set(we1f[1])
    wd = wd.at[2].set(we1f[3]).at[3].set(we1f[4])
    wd = wd.at[4].set(we1f[6])
    wdist = jnp.stack([we1f[2], we1f[5]], axis=0)
    ssel = jnp.zeros((16, 2), f32)
    ssel = ssel.at[0, 0].set(1.0).at[1, 0].set(1.0)
    ssel = ssel.at[2, 1].set(1.0).at[3, 1].set(1.0)

    x_h = _tc_encode(x, wn1f, _row(bn1f), wn2, _row(bn2), _row(gn), _row(ben),
                     bn=2048)

    pr, pc = _sc_gather2(ptab, rowp, colp, 16)
    e_h = _tc_edge_encode(pr, pc, ssel, wd, wdist, _row(be1f), we2, _row(be2),
                          _row(ge), _row(bee), bn=2048)

    zeros_np = jnp.zeros((NP, H), f32)

    for proc in params["processors"]:
        (pw1, pb1), (pw2, pb2) = proc["edge_mlp"]["layers"]
        pg, pbe = proc["edge_mlp"]["ln"]
        w1a = pw1[0:H]
        w1b = pw1[H:2 * H]
        w1c = pw1[2 * H:3 * H]
        w1ab = jnp.concatenate([w1a, w1b], axis=1)

        (nw1, nb1), (nw2, nb2) = proc["node_mlp"]["layers"]
        ng, nbe = proc["node_mlp"]["ln"]
        wna = nw1[0:H]
        wnb = nw1[H:2 * H]

        xpack = lax.bitcast_convert_type(
            x_h.astype(jnp.bfloat16).reshape(NP, H // 2, 2), jnp.float32)
        xrp, xcp = _sc_gather2(xpack, rowp, colp, H // 2)
        xr = lax.bitcast_convert_type(xrp, jnp.bfloat16).reshape(EP, H)
        xc = lax.bitcast_convert_type(xcp, jnp.bfloat16).reshape(EP, H)
        msg, e_h = _tc_edge_step(xr, xc, e_h, w1ab.astype(jnp.bfloat16),
                                 w1c.astype(jnp.bfloat16), _row(pb1),
                                 pw2.astype(jnp.bfloat16),
                                 _row(pb2), _row(pg), _row(pbe), bn=1024)
        a0, a1 = _sc_scatter_add(msg, colp, zeros_np)
        x_h = _tc_node_step(a0, a1, x_h, wna, wnb, _row(nb1), nw2, _row(nb2),
                            _row(ng), _row(nbe), bn=2048)

    (dw1, db1), (dw2, db2) = params["node_decoder"]["layers"]
    dw2p = jnp.zeros((H, H), f32).at[:, :dw2.shape[1]].set(dw2)
    db2p = jnp.zeros((H,), f32).at[:dw2.shape[1]].set(db2)
    out = _tc_decode(x_h, dw1, _row(db1), dw2p, _row(db2p), bn=2048)
    return out[:N, :dw2.shape[1]]


# speedup vs baseline: 1.8778x; 1.8778x over previous
import functools

import jax
import jax.numpy as jnp
from jax import lax
from jax.experimental import pallas as pl
from jax.experimental.pallas import tpu as pltpu
from jax.experimental.pallas import tpu_sc as plsc

N = 10000
E = 320000
H = 128

NC = 2
NS = 16
NW = NC * NS

NP = 10240
EP = 327680
PER_W = EP // NW
C = 128
NCHUNK = PER_W // C
ROWS_PER_SUB = NP // NS

def _sc_mesh():
    return plsc.VectorSubcoreMesh(core_axis_name="c", subcore_axis_name="s",
                                  num_cores=NC, num_subcores=NS)


def _sc_gather2(table, rowp, colp, d):
    kk = 80
    tot = (EP // C) // NS
    kmax = max(kk, tot - kk)

    def body(tab, rid, cid, out_r, out_c, idxs, b00, b01, b10, b11,
             g00, g01, g10, g11, w00, w01, w10, w11):
        cor = lax.axis_index("c")
        sid = lax.axis_index("s")

        bufs = ((b00, b01), (b10, b11))
        gsem = ((g00, g01), (g10, g11))
        wsem = ((w00, w01), (w10, w11))
        outs = (out_r, out_c)

        def run(nch, base):
            pltpu.sync_copy(rid.at[pl.ds(base, nch * C)],
                            idxs.at[0, pl.ds(0, nch * C)])
            pltpu.sync_copy(cid.at[pl.ds(base, nch * C)],
                            idxs.at[1, pl.ds(0, nch * C)])

            def fire(b, p, g):
                pltpu.async_copy(
                    tab.at[idxs.at[b, pl.ds(g * C, C)]], bufs[b][p],
                    gsem[b][p])

            def wait_gather(b, p):
                pltpu.make_async_copy(
                    tab.at[pl.ds(0, C)], bufs[b][p], gsem[b][p]).wait()

            def wait_wb(b, p):
                pltpu.make_async_copy(
                    bufs[b][p], outs[b].at[pl.ds(base, C)], wsem[b][p]).wait()

            for b in (0, 1):
                fire(b, 0, 0)

            @pl.loop(0, nch, step=2)
            def _(g0):
                for p in (0, 1):
                    g = g0 + p
                    for b in (0, 1):
                        wait_gather(b, p)
                        pltpu.async_copy(
                            bufs[b][p], outs[b].at[pl.ds(base + g * C, C)],
                            wsem[b][p])

                        @pl.when(g + 1 < nch)
                        def _():
                            @pl.when(g >= 1)
                            def _():
                                wait_wb(b, 1 - p)
                            fire(b, 1 - p, g + 1)

            for b in (0, 1):
                wait_wb(b, (nch - 1) % 2)

        @pl.when(cor == 0)
        def _():
            run(kk, sid * (kk * C))

        @pl.when(cor == 1)
        def _():
            run(tot - kk, (NS * kk + sid * (tot - kk)) * C)

    fn = pl.kernel(
        body,
        out_type=[
            jax.ShapeDtypeStruct((EP, d), jnp.float32),
            jax.ShapeDtypeStruct((EP, d), jnp.float32),
        ],
        mesh=_sc_mesh(),
        scratch_types=[
            pltpu.VMEM((2, kmax * C), jnp.int32),
            pltpu.VMEM((C, d), jnp.float32),
            pltpu.VMEM((C, d), jnp.float32),
            pltpu.VMEM((C, d), jnp.float32),
            pltpu.VMEM((C, d), jnp.float32),
            pltpu.SemaphoreType.DMA,
            pltpu.SemaphoreType.DMA,
            pltpu.SemaphoreType.DMA,
            pltpu.SemaphoreType.DMA,
            pltpu.SemaphoreType.DMA,
            pltpu.SemaphoreType.DMA,
            pltpu.SemaphoreType.DMA,
            pltpu.SemaphoreType.DMA,
        ],
        name=f"sc_gather2_{d}",
        compiler_params=pltpu.CompilerParams(use_tc_tiling_on_sc=(d % 128 == 0)),
    )
    return fn(table, rowp, colp)


def _sc_scatter_add(msg, colp, zeros_np):
    def body(m, cidx, zer, out0, out1, idx, buf, acc, sem):
        cid = lax.axis_index("c")
        sid = lax.axis_index("s")
        myrows = pl.ds(sid * ROWS_PER_SUB, ROWS_PER_SUB)
        pltpu.sync_copy(zer.at[myrows], acc.at[myrows])
        plsc.subcore_barrier()

        base = (sid * NC + cid) * PER_W

        @pl.loop(0, NCHUNK)
        def _(k):
            off = base + k * C
            pltpu.sync_copy(cidx.at[pl.ds(off, C)], idx)
            cp = pltpu.async_copy(m.at[pl.ds(off, C)], buf, sem)
            cp.wait()
            pltpu.sync_copy(buf, acc.at[idx], add=True)

        plsc.subcore_barrier()

        @pl.when(cid == 0)
        def _():
            pltpu.sync_copy(acc.at[myrows], out0.at[myrows])

        @pl.when(cid == 1)
        def _():
            pltpu.sync_copy(acc.at[myrows], out1.at[myrows])

    fn = pl.kernel(
        body,
        out_type=[
            jax.ShapeDtypeStruct((NP, H), jnp.float32),
            jax.ShapeDtypeStruct((NP, H), jnp.float32),
        ],
        mesh=_sc_mesh(),
        scratch_types=[
            pltpu.VMEM((C,), jnp.int32),
            pltpu.VMEM((C, H), jnp.float32),
            pltpu.VMEM_SHARED((NP, H), jnp.float32),
            pltpu.SemaphoreType.DMA,
        ],
        name="sc_scatter_add",
    )
    return fn(msg, colp, zeros_np)


def _ln(y, g, be):
    mu = jnp.mean(y, axis=-1, keepdims=True)
    var = jnp.mean((y - mu) * (y - mu), axis=-1, keepdims=True)
    return (y - mu) * lax.rsqrt(var + 1e-5) * g + be


def _full(shape):
    return pl.BlockSpec(shape, lambda i: (0,) * len(shape))


def _rows(b, d):
    return pl.BlockSpec((b, d), lambda i: (i, 0))


def _pack_bf16(o):
    u = lax.bitcast_convert_type(o, jnp.uint32)
    r = u + jnp.uint32(0x7FFF) + ((u >> 16) & jnp.uint32(1))
    lo = r[:, :H // 2] >> 16
    hi = r[:, H // 2:] & jnp.uint32(0xFFFF0000)
    return lax.bitcast_convert_type(hi | lo, jnp.float32)


def _unpack_bf16(p):
    u = lax.bitcast_convert_type(p, jnp.uint32)
    lo = lax.bitcast_convert_type(u << 16, jnp.float32)
    hi = lax.bitcast_convert_type(u & jnp.uint32(0xFFFF0000), jnp.float32)
    return jnp.concatenate([lo, hi], axis=1)


def _encode_body(x_ref, w1, b1, w2, b2, g, be, o_ref, op_ref):
    h = jnp.maximum(x_ref[...] @ w1[...] + b1[...], 0.0)
    o = _ln(h @ w2[...] + b2[...], g[...], be[...])
    o_ref[...] = o
    op_ref[...] = _pack_bf16(o)


def _tc_encode(x, w1, b1, w2, b2, g, be, bn):
    n, din = x.shape
    return pl.pallas_call(
        _encode_body,
        grid=(n // bn,),
        in_specs=[
            _rows(bn, din),
            _full(w1.shape), _full(b1.shape), _full(w2.shape),
            _full(b2.shape), _full(g.shape), _full(be.shape),
        ],
        out_specs=[_rows(bn, H), _rows(bn, H // 2)],
        out_shape=[
            jax.ShapeDtypeStruct((n, H), jnp.float32),
            jax.ShapeDtypeStruct((n, H // 2), jnp.float32),
        ],
        name="tc_encode",
    )(x, w1, b1, w2, b2, g, be)


def _edge_enc_body(pr_ref, pc_ref, ssel, wd, wdist, b1, w2, b2, g, be, o_ref):
    d = pr_ref[...] - pc_ref[...]
    s2 = (d * d) @ ssel[...]
    dist = jnp.sqrt(s2 + 1e-12)
    h = jnp.maximum(d @ wd[...] + dist @ wdist[...] + b1[...], 0.0)
    o_ref[...] = _ln(h @ w2[...] + b2[...], g[...], be[...])


def _tc_edge_encode(pr, pc, ssel, wd, wdist, b1, w2, b2, g, be, bn):
    return pl.pallas_call(
        _edge_enc_body,
        grid=(EP // bn,),
        in_specs=[
            _rows(bn, pr.shape[1]), _rows(bn, pc.shape[1]),
            _full(ssel.shape), _full(wd.shape), _full(wdist.shape),
            _full(b1.shape), _full(w2.shape), _full(b2.shape),
            _full(g.shape), _full(be.shape),
        ],
        out_specs=_rows(bn, H),
        out_shape=jax.ShapeDtypeStruct((EP, H), jnp.float32),
        name="tc_edge_encode",
    )(pr, pc, ssel, wd, wdist, b1, w2, b2, g, be)


def _dot(a, b):
    return jnp.dot(a, b, preferred_element_type=jnp.float32)


def _edge_step_body(xr_ref, xc_ref, eh_ref, w1ab, w1c, b1, w2, b2, g, be,
                    msg_ref, ne_ref):
    eh = eh_ref[...]
    xr = _unpack_bf16(xr_ref[...])
    xc = _unpack_bf16(xc_ref[...])
    mc = _dot(xc, w1ab[...])
    mr = _dot(xr, w1ab[...])
    ec = _dot(eh, w1c[...]) + b1[...]
    pre_m = mc[:, :H] + mr[:, H:] + ec
    pre_n = mr[:, :H] + mc[:, H:] + ec
    hm = jnp.maximum(pre_m, 0.0)
    hn = jnp.maximum(pre_n, 0.0)
    msg_ref[...] = _ln(_dot(hm, w2[...]) + b2[...], g[...], be[...])
    ne_ref[...] = eh + _ln(_dot(hn, w2[...]) + b2[...], g[...], be[...])


def _tc_edge_step(xr, xc, eh, w1ab, w1c, b1, w2, b2, g, be, bn):
    return pl.pallas_call(
        _edge_step_body,
        grid=(EP // bn,),
        in_specs=[
            _rows(bn, H // 2), _rows(bn, H // 2), _rows(bn, H),
            _full(w1ab.shape), _full(w1c.shape), _full(b1.shape),
            _full(w2.shape), _full(b2.shape), _full(g.shape), _full(be.shape),
        ],
        out_specs=[_rows(bn, H), _rows(bn, H)],
        out_shape=[
            jax.ShapeDtypeStruct((EP, H), jnp.float32),
            jax.ShapeDtypeStruct((EP, H), jnp.float32),
        ],
        name="tc_edge_step",
    )(xr, xc, eh, w1ab, w1c, b1, w2, b2, g, be)


def _node_step_body(a0_ref, a1_ref, xh_ref, wna, wnb, b1, w2, b2, g, be,
                    o_ref, op_ref):
    xh = xh_ref[...]
    pre = (a0_ref[...] + a1_ref[...]) @ wna[...] + xh @ wnb[...] + b1[...]
    h = jnp.maximum(pre, 0.0)
    o = xh + _ln(h @ w2[...] + b2[...], g[...], be[...])
    o_ref[...] = o
    op_ref[...] = _pack_bf16(o)


def _tc_node_step(a0, a1, xh, wna, wnb, b1, w2, b2, g, be, bn):
    return pl.pallas_call(
        _node_step_body,
        grid=(NP // bn,),
        in_specs=[
            _rows(bn, H), _rows(bn, H), _rows(bn, H),
            _full(wna.shape), _full(wnb.shape), _full(b1.shape),
            _full(w2.shape), _full(b2.shape), _full(g.shape), _full(be.shape),
        ],
        out_specs=[_rows(bn, H), _rows(bn, H // 2)],
        out_shape=[
            jax.ShapeDtypeStruct((NP, H), jnp.float32),
            jax.ShapeDtypeStruct((NP, H // 2), jnp.float32),
        ],
        name="tc_node_step",
    )(a0, a1, xh, wna, wnb, b1, w2, b2, g, be)


def _decode_body(xh_ref, w1, b1, w2, b2, o_ref):
    h = jnp.maximum(xh_ref[...] @ w1[...] + b1[...], 0.0)
    o_ref[...] = h @ w2[...] + b2[...]


def _tc_decode(xh, w1, b1, w2, b2, bn):
    return pl.pallas_call(
        _decode_body,
        grid=(NP // bn,),
        in_specs=[
            _rows(bn, H),
            _full(w1.shape), _full(b1.shape), _full(w2.shape), _full(b2.shape),
        ],
        out_specs=_rows(bn, H),
        out_shape=jax.ShapeDtypeStruct((NP, H), jnp.float32),
        name="tc_decode",
    )(xh, w1, b1, w2, b2)


def _fold_norm(w1, b1, mu, sig):
    w1f = w1 / sig[:, None]
    b1f = b1 - (mu / sig) @ w1
    return w1f, b1f


def _row(v):
    return v.reshape(1, -1)


def kernel(world_pos, mesh_pos, phi, swelling_phi, swelling_phi_rate,
           node_type, time, mat_param, edge_index, params):
    f32 = jnp.float32

    u = world_pos - mesh_pos
    freqs = 2.0 ** jnp.arange(2, dtype=f32)
    temb = jnp.concatenate([jnp.sin(freqs * time), jnp.cos(freqs * time)])
    x = jnp.concatenate(
        [u, phi, swelling_phi, swelling_phi_rate, node_type,
         jnp.tile(temb[None, :], (N, 1)), jnp.tile(mat_param[None, :], (N, 1))],
        axis=-1)
    x = jnp.pad(x, ((0, NP - N), (0, 0)))

    ptab = jnp.concatenate(
        [mesh_pos, world_pos, phi, jnp.zeros((N, 11), f32)], axis=-1)
    ptab = jnp.pad(ptab, ((0, NP - N), (0, 0)))

    rowp = jnp.full((EP,), N, jnp.int32).at[:E].set(edge_index[0])
    colp = jnp.full((EP,), N, jnp.int32).at[:E].set(edge_index[1])

    nmu, nsig = params["node_norm"]
    (wn1, bn1), (wn2, bn2) = params["node_encoder"]["layers"]
    gn, ben = params["node_encoder"]["ln"]
    wn1f, bn1f = _fold_norm(wn1, bn1, nmu, nsig)

    emu, esig = params["edge_norm"]
    (we1, be1), (we2, be2) = params["edge_encoder"]["layers"]
    ge, bee = params["edge_encoder"]["ln"]
    we1f, be1f = _fold_norm(we1, be1, emu, esig)
    wd = jnp.zeros((16, H), f32)
    wd = wd.at[0].set(we1f[0]).at[1].set(we1f[1])
    wd = wd.at[2].set(we1f[3]).at[3].set(we1f[4])
    wd = wd.at[4].set(we1f[6])
    wdist = jnp.stack([we1f[2], we1f[5]], axis=0)
    ssel = jnp.zeros((16, 2), f32)
    ssel = ssel.at[0, 0].set(1.0).at[1, 0].set(1.0)
    ssel = ssel.at[2, 1].set(1.0).at[3, 1].set(1.0)

    x_h, xpk = _tc_encode(x, wn1f, _row(bn1f), wn2, _row(bn2), _row(gn),
                          _row(ben), bn=2048)

    pr, pc = _sc_gather2(ptab, rowp, colp, 16)
    e_h = _tc_edge_encode(pr, pc, ssel, wd, wdist, _row(be1f), we2, _row(be2),
                          _row(ge), _row(bee), bn=2048)

    zeros_np = jnp.zeros((NP, H), f32)

    for proc in params["processors"]:
        (pw1, pb1), (pw2, pb2) = proc["edge_mlp"]["layers"]
        pg, pbe = proc["edge_mlp"]["ln"]
        w1a = pw1[0:H]
        w1b = pw1[H:2 * H]
        w1c = pw1[2 * H:3 * H]
        w1ab = jnp.concatenate([w1a, w1b], axis=1)

        (nw1, nb1), (nw2, nb2) = proc["node_mlp"]["layers"]
        ng, nbe = proc["node_mlp"]["ln"]
        wna = nw1[0:H]
        wnb = nw1[H:2 * H]

        xrp, xcp = _sc_gather2(xpk, rowp, colp, H // 2)
        msg, e_h = _tc_edge_step(xrp, xcp, e_h, w1ab, w1c, _row(pb1), pw2,
                                 _row(pb2), _row(pg), _row(pbe), bn=1024)
        a0, a1 = _sc_scatter_add(msg, colp, zeros_np)
        x_h, xpk = _tc_node_step(a0, a1, x_h, wna, wnb, _row(nb1), nw2,
                                 _row(nb2), _row(ng), _row(nbe), bn=2048)

    (dw1, db1), (dw2, db2) = params["node_decoder"]["layers"]
    dw2p = jnp.zeros((H, H), f32).at[:, :dw2.shape[1]].set(dw2)
    db2p = jnp.zeros((H,), f32).at[:dw2.shape[1]].set(db2)
    out = _tc_decode(x_h, dw1, _row(db1), dw2p, _row(db2p), bn=2048)
    return out[:N, :dw2.shape[1]]


# speedup vs baseline: 2.1169x; 1.1273x over previous
import functools

import jax
import jax.numpy as jnp
from jax import lax
from jax.experimental import pallas as pl
from jax.experimental.pallas import tpu as pltpu
from jax.experimental.pallas import tpu_sc as plsc

N = 10000
E = 320000
H = 128

NC = 2
NS = 16
NW = NC * NS

NP = 10240
EP = 327680
PER_W = EP // NW
C = 128
NCHUNK = PER_W // C
ROWS_PER_SUB = NP // NS

def _sc_mesh():
    return plsc.VectorSubcoreMesh(core_axis_name="c", subcore_axis_name="s",
                                  num_cores=NC, num_subcores=NS)


def _sc_gather2(table, rowp, colp, d):
    kk = 160 if d == H // 2 else 80
    tot = (EP // C) // NS
    kmax = max(kk, tot - kk)

    def body(tab, rid, cid, out_r, out_c, idxs, b00, b01, b10, b11,
             g00, g01, g10, g11, w00, w01, w10, w11):
        cor = lax.axis_index("c")
        sid = lax.axis_index("s")

        bufs = ((b00, b01), (b10, b11))
        gsem = ((g00, g01), (g10, g11))
        wsem = ((w00, w01), (w10, w11))
        outs = (out_r, out_c)

        def run(nch, base):
            if nch == 0:
                return
            pltpu.sync_copy(rid.at[pl.ds(base, nch * C)],
                            idxs.at[0, pl.ds(0, nch * C)])
            pltpu.sync_copy(cid.at[pl.ds(base, nch * C)],
                            idxs.at[1, pl.ds(0, nch * C)])

            def fire(b, p, g):
                pltpu.async_copy(
                    tab.at[idxs.at[b, pl.ds(g * C, C)]], bufs[b][p],
                    gsem[b][p])

            def wait_gather(b, p):
                pltpu.make_async_copy(
                    tab.at[pl.ds(0, C)], bufs[b][p], gsem[b][p]).wait()

            def wait_wb(b, p):
                pltpu.make_async_copy(
                    bufs[b][p], outs[b].at[pl.ds(base, C)], wsem[b][p]).wait()

            for b in (0, 1):
                fire(b, 0, 0)

            @pl.loop(0, nch, step=2)
            def _(g0):
                for p in (0, 1):
                    g = g0 + p
                    for b in (0, 1):
                        wait_gather(b, p)
                        pltpu.async_copy(
                            bufs[b][p], outs[b].at[pl.ds(base + g * C, C)],
                            wsem[b][p])

                        @pl.when(g + 1 < nch)
                        def _():
                            @pl.when(g >= 1)
                            def _():
                                wait_wb(b, 1 - p)
                            fire(b, 1 - p, g + 1)

            for b in (0, 1):
                wait_wb(b, (nch - 1) % 2)

        @pl.when(cor == 0)
        def _():
            run(kk, sid * (kk * C))

        @pl.when(cor == 1)
        def _():
            run(tot - kk, (NS * kk + sid * (tot - kk)) * C)

    fn = pl.kernel(
        body,
        out_type=[
            jax.ShapeDtypeStruct((EP, d), jnp.float32),
            jax.ShapeDtypeStruct((EP, d), jnp.float32),
        ],
        mesh=_sc_mesh(),
        scratch_types=[
            pltpu.VMEM((2, kmax * C), jnp.int32),
            pltpu.VMEM((C, d), jnp.float32),
            pltpu.VMEM((C, d), jnp.float32),
            pltpu.VMEM((C, d), jnp.float32),
            pltpu.VMEM((C, d), jnp.float32),
            pltpu.SemaphoreType.DMA,
            pltpu.SemaphoreType.DMA,
            pltpu.SemaphoreType.DMA,
            pltpu.SemaphoreType.DMA,
            pltpu.SemaphoreType.DMA,
            pltpu.SemaphoreType.DMA,
            pltpu.SemaphoreType.DMA,
            pltpu.SemaphoreType.DMA,
        ],
        name=f"sc_gather2_{d}",
        compiler_params=pltpu.CompilerParams(use_tc_tiling_on_sc=(d % 128 == 0)),
    )
    return fn(table, rowp, colp)


HALF_E = EP // 2
PER_S = HALF_E // NS


def _sc_scatter_add(msga, msgb, colp, zeros_np):
    def body(ma, mb, cidx, zer, out0, out1, idx, buf, acc, sem):
        cid = lax.axis_index("c")
        sid = lax.axis_index("s")
        myrows = pl.ds(sid * ROWS_PER_SUB, ROWS_PER_SUB)
        pltpu.sync_copy(zer.at[myrows], acc.at[myrows])
        plsc.subcore_barrier()

        def run(m, half_base):
            base = half_base + sid * PER_S

            @pl.loop(0, PER_S // C)
            def _(k):
                off = base + k * C
                pltpu.sync_copy(cidx.at[pl.ds(off, C)], idx)
                cp = pltpu.async_copy(m.at[pl.ds(off - half_base, C)], buf,
                                      sem)
                cp.wait()
                pltpu.sync_copy(buf, acc.at[idx], add=True)

        @pl.when(cid == 0)
        def _():
            run(ma, 0)

        @pl.when(cid == 1)
        def _():
            run(mb, HALF_E)

        plsc.subcore_barrier()

        @pl.when(cid == 0)
        def _():
            pltpu.sync_copy(acc.at[myrows], out0.at[myrows])

        @pl.when(cid == 1)
        def _():
            pltpu.sync_copy(acc.at[myrows], out1.at[myrows])

    fn = pl.kernel(
        body,
        out_type=[
            jax.ShapeDtypeStruct((NP, H), jnp.float32),
            jax.ShapeDtypeStruct((NP, H), jnp.float32),
        ],
        mesh=_sc_mesh(),
        scratch_types=[
            pltpu.VMEM((C,), jnp.int32),
            pltpu.VMEM((C, H), jnp.float32),
            pltpu.VMEM_SHARED((NP, H), jnp.float32),
            pltpu.SemaphoreType.DMA,
        ],
        name="sc_scatter_add",
    )
    return fn(msga, msgb, colp, zeros_np)


def _ln(y, g, be):
    mu = jnp.mean(y, axis=-1, keepdims=True)
    var = jnp.mean((y - mu) * (y - mu), axis=-1, keepdims=True)
    return (y - mu) * lax.rsqrt(var + 1e-5) * g + be


def _full(shape):
    return pl.BlockSpec(shape, lambda i: (0,) * len(shape))


def _rows(b, d):
    return pl.BlockSpec((b, d), lambda i: (i, 0))


def _pack_bf16(o):
    u = lax.bitcast_convert_type(o, jnp.uint32)
    r = u + jnp.uint32(0x7FFF) + ((u >> 16) & jnp.uint32(1))
    lo = r[:, :H // 2] >> 16
    hi = r[:, H // 2:] & jnp.uint32(0xFFFF0000)
    return lax.bitcast_convert_type(hi | lo, jnp.float32)


def _unpack_bf16(p):
    u = lax.bitcast_convert_type(p, jnp.uint32)
    lo = lax.bitcast_convert_type(u << 16, jnp.float32)
    hi = lax.bitcast_convert_type(u & jnp.uint32(0xFFFF0000), jnp.float32)
    return jnp.concatenate([lo, hi], axis=1)


def _encode_body(x_ref, w1, b1, w2, b2, g, be, o_ref, op_ref):
    h = jnp.maximum(x_ref[...] @ w1[...] + b1[...], 0.0)
    o = _ln(h @ w2[...] + b2[...], g[...], be[...])
    o_ref[...] = o
    op_ref[...] = _pack_bf16(o)


def _tc_encode(x, w1, b1, w2, b2, g, be, bn):
    n, din = x.shape
    return pl.pallas_call(
        _encode_body,
        grid=(n // bn,),
        in_specs=[
            _rows(bn, din),
            _full(w1.shape), _full(b1.shape), _full(w2.shape),
            _full(b2.shape), _full(g.shape), _full(be.shape),
        ],
        out_specs=[_rows(bn, H), _rows(bn, H // 2)],
        out_shape=[
            jax.ShapeDtypeStruct((n, H), jnp.float32),
            jax.ShapeDtypeStruct((n, H // 2), jnp.float32),
        ],
        name="tc_encode",
    )(x, w1, b1, w2, b2, g, be)


def _edge_enc_body(pra_ref, prb_ref, pca_ref, pcb_ref, ssel, wd, wdist, b1,
                   w2, b2, g, be, oa_ref, ob_ref):
    pr = jnp.concatenate([pra_ref[...], prb_ref[...]], axis=0)
    pc = jnp.concatenate([pca_ref[...], pcb_ref[...]], axis=0)
    d = pr - pc
    s2 = (d * d) @ ssel[...]
    dist = jnp.sqrt(s2 + 1e-12)
    h = jnp.maximum(d @ wd[...] + dist @ wdist[...] + b1[...], 0.0)
    o = _ln(h @ w2[...] + b2[...], g[...], be[...])
    bn2 = oa_ref.shape[0]
    oa_ref[...] = o[:bn2]
    ob_ref[...] = o[bn2:]


def _tc_edge_encode(pr, pc, ssel, wd, wdist, b1, w2, b2, g, be, bn):
    bn2 = bn // 2
    nblk = HALF_E // bn2
    dp = pr.shape[1]

    def rows_a(i):
        return (i, 0)

    def rows_b(i):
        return (i + nblk, 0)

    return pl.pallas_call(
        _edge_enc_body,
        grid=(nblk,),
        in_specs=[
            pl.BlockSpec((bn2, dp), rows_a), pl.BlockSpec((bn2, dp), rows_b),
            pl.BlockSpec((bn2, dp), rows_a), pl.BlockSpec((bn2, dp), rows_b),
            _full(ssel.shape), _full(wd.shape), _full(wdist.shape),
            _full(b1.shape), _full(w2.shape), _full(b2.shape),
            _full(g.shape), _full(be.shape),
        ],
        out_specs=[_rows(bn2, H), _rows(bn2, H)],
        out_shape=[
            jax.ShapeDtypeStruct((HALF_E, H), jnp.float32),
            jax.ShapeDtypeStruct((HALF_E, H), jnp.float32),
        ],
        name="tc_edge_encode",
    )(pr, pr, pc, pc, ssel, wd, wdist, b1, w2, b2, g, be)


def _dot(a, b):
    return jnp.dot(a, b, preferred_element_type=jnp.float32)


def _unpack_pair(p):
    u = lax.bitcast_convert_type(p, jnp.uint32)
    lo = lax.bitcast_convert_type(u << 16, jnp.float32)
    hi = lax.bitcast_convert_type(u & jnp.uint32(0xFFFF0000), jnp.float32)
    d2 = H // 2
    xa = jnp.concatenate([lo[:, :d2], hi[:, :d2]], axis=1)
    xb = jnp.concatenate([lo[:, d2:], hi[:, d2:]], axis=1)
    return jnp.concatenate([xa, xb], axis=0)


def _edge_step_body(xr_ref, xc_ref, eha_ref, ehb_ref, w1ab, w1c, b1, w2, b2,
                    g, be, msga_ref, msgb_ref, nea_ref, neb_ref):
    eh = jnp.concatenate([eha_ref[...], ehb_ref[...]], axis=0)
    xr = _unpack_pair(xr_ref[...])
    xc = _unpack_pair(xc_ref[...])
    mc = _dot(xc, w1ab[...])
    mr = _dot(xr, w1ab[...])
    ec = _dot(eh, w1c[...]) + b1[...]
    pre_m = mc[:, :H] + mr[:, H:] + ec
    pre_n = mr[:, :H] + mc[:, H:] + ec
    hm = jnp.maximum(pre_m, 0.0)
    hn = jnp.maximum(pre_n, 0.0)
    msg = _ln(_dot(hm, w2[...]) + b2[...], g[...], be[...])
    ne = eh + _ln(_dot(hn, w2[...]) + b2[...], g[...], be[...])
    bn2 = msga_ref.shape[0]
    msga_ref[...] = msg[:bn2]
    msgb_ref[...] = msg[bn2:]
    nea_ref[...] = ne[:bn2]
    neb_ref[...] = ne[bn2:]


def _tc_edge_step(xr, xc, eha, ehb, w1ab, w1c, b1, w2, b2, g, be, bn):
    bn2 = bn // 2
    half_shape = jax.ShapeDtypeStruct((HALF_E, H), jnp.float32)
    return pl.pallas_call(
        _edge_step_body,
        grid=(HALF_E // bn2,),
        in_specs=[
            _rows(bn2, H), _rows(bn2, H), _rows(bn2, H), _rows(bn2, H),
            _full(w1ab.shape), _full(w1c.shape), _full(b1.shape),
            _full(w2.shape), _full(b2.shape), _full(g.shape), _full(be.shape),
        ],
        out_specs=[_rows(bn2, H)] * 4,
        out_shape=[half_shape] * 4,
        name="tc_edge_step",
    )(xr, xc, eha, ehb, w1ab, w1c, b1, w2, b2, g, be)


def _node_step_body(a0_ref, a1_ref, xh_ref, wna, wnb, b1, w2, b2, g, be,
                    o_ref, op_ref):
    xh = xh_ref[...]
    pre = (a0_ref[...] + a1_ref[...]) @ wna[...] + xh @ wnb[...] + b1[...]
    h = jnp.maximum(pre, 0.0)
    o = xh + _ln(h @ w2[...] + b2[...], g[...], be[...])
    o_ref[...] = o
    op_ref[...] = _pack_bf16(o)


def _tc_node_step(a0, a1, xh, wna, wnb, b1, w2, b2, g, be, bn):
    return pl.pallas_call(
        _node_step_body,
        grid=(NP // bn,),
        in_specs=[
            _rows(bn, H), _rows(bn, H), _rows(bn, H),
            _full(wna.shape), _full(wnb.shape), _full(b1.shape),
            _full(w2.shape), _full(b2.shape), _full(g.shape), _full(be.shape),
        ],
        out_specs=[_rows(bn, H), _rows(bn, H // 2)],
        out_shape=[
            jax.ShapeDtypeStruct((NP, H), jnp.float32),
            jax.ShapeDtypeStruct((NP, H // 2), jnp.float32),
        ],
        name="tc_node_step",
    )(a0, a1, xh, wna, wnb, b1, w2, b2, g, be)


def _decode_body(xh_ref, w1, b1, w2, b2, o_ref):
    h = jnp.maximum(xh_ref[...] @ w1[...] + b1[...], 0.0)
    o_ref[...] = h @ w2[...] + b2[...]


def _tc_decode(xh, w1, b1, w2, b2, bn):
    return pl.pallas_call(
        _decode_body,
        grid=(NP // bn,),
        in_specs=[
            _rows(bn, H),
            _full(w1.shape), _full(b1.shape), _full(w2.shape), _full(b2.shape),
        ],
        out_specs=_rows(bn, H),
        out_shape=jax.ShapeDtypeStruct((NP, H), jnp.float32),
        name="tc_decode",
    )(xh, w1, b1, w2, b2)


def _fold_norm(w1, b1, mu, sig):
    w1f = w1 / sig[:, None]
    b1f = b1 - (mu / sig) @ w1
    return w1f, b1f


def _row(v):
    return v.reshape(1, -1)


def kernel(world_pos, mesh_pos, phi, swelling_phi, swelling_phi_rate,
           node_type, time, mat_param, edge_index, params):
    f32 = jnp.float32

    u = world_pos - mesh_pos
    freqs = 2.0 ** jnp.arange(2, dtype=f32)
    temb = jnp.concatenate([jnp.sin(freqs * time), jnp.cos(freqs * time)])
    x = jnp.concatenate(
        [u, phi, swelling_phi, swelling_phi_rate, node_type,
         jnp.tile(temb[None, :], (N, 1)), jnp.tile(mat_param[None, :], (N, 1))],
        axis=-1)
    x = jnp.pad(x, ((0, NP - N), (0, 0)))

    ptab = jnp.concatenate(
        [mesh_pos, world_pos, phi, jnp.zeros((N, 11), f32)], axis=-1)
    ptab = jnp.pad(ptab, ((0, NP - N), (0, 0)))

    rowp = jnp.full((EP,), N, jnp.int32).at[:E].set(edge_index[0])
    colp = jnp.full((EP,), N, jnp.int32).at[:E].set(edge_index[1])
    rowi = jnp.stack([rowp[:EP // 2], rowp[EP // 2:]], axis=1).reshape(EP)
    coli = jnp.stack([colp[:EP // 2], colp[EP // 2:]], axis=1).reshape(EP)

    nmu, nsig = params["node_norm"]
    (wn1, bn1), (wn2, bn2) = params["node_encoder"]["layers"]
    gn, ben = params["node_encoder"]["ln"]
    wn1f, bn1f = _fold_norm(wn1, bn1, nmu, nsig)

    emu, esig = params["edge_norm"]
    (we1, be1), (we2, be2) = params["edge_encoder"]["layers"]
    ge, bee = params["edge_encoder"]["ln"]
    we1f, be1f = _fold_norm(we1, be1, emu, esig)
    wd = jnp.zeros((16, H), f32)
    wd = wd.at[0].set(we1f[0]).at[1].set(we1f[1])
    wd = wd.at[2].set(we1f[3]).at[3].set(we1f[4])
    wd = wd.at[4].set(we1f[6])
    wdist = jnp.stack([we1f[2], we1f[5]], axis=0)
    ssel = jnp.zeros((16, 2), f32)
    ssel = ssel.at[0, 0].set(1.0).at[1, 0].set(1.0)
    ssel = ssel.at[2, 1].set(1.0).at[3, 1].set(1.0)

    x_h, xpk = _tc_encode(x, wn1f, _row(bn1f), wn2, _row(bn2), _row(gn),
                          _row(ben), bn=2048)

    pr, pc = _sc_gather2(ptab, rowp, colp, 16)
    eha, ehb = _tc_edge_encode(pr, pc, ssel, wd, wdist, _row(be1f), we2,
                               _row(be2), _row(ge), _row(bee), bn=2048)

    zeros_np = jnp.zeros((NP, H), f32)

    for proc in params["processors"]:
        (pw1, pb1), (pw2, pb2) = proc["edge_mlp"]["layers"]
        pg, pbe = proc["edge_mlp"]["ln"]
        w1a = pw1[0:H]
        w1b = pw1[H:2 * H]
        w1c = pw1[2 * H:3 * H]
        w1ab = jnp.concatenate([w1a, w1b], axis=1)

        (nw1, nb1), (nw2, nb2) = proc["node_mlp"]["layers"]
        ng, nbe = proc["node_mlp"]["ln"]
        wna = nw1[0:H]
        wnb = nw1[H:2 * H]

        xr64, xc64 = _sc_gather2(xpk, rowi, coli, H // 2)
        xrp = xr64.reshape(HALF_E, H)
        xcp = xc64.reshape(HALF_E, H)
        msga, msgb, eha, ehb = _tc_edge_step(
            xrp, xcp, eha, ehb, w1ab, w1c, _row(pb1), pw2, _row(pb2),
            _row(pg), _row(pbe), bn=1024)
        a0, a1 = _sc_scatter_add(msga, msgb, colp, zeros_np)
        x_h, xpk = _tc_node_step(a0, a1, x_h, wna, wnb, _row(nb1), nw2,
                                 _row(nb2), _row(ng), _row(nbe), bn=2048)

    (dw1, db1), (dw2, db2) = params["node_decoder"]["layers"]
    dw2p = jnp.zeros((H, H), f32).at[:, :dw2.shape[1]].set(dw2)
    db2p = jnp.zeros((H,), f32).at[:dw2.shape[1]].set(db2)
    out = _tc_decode(x_h, dw1, _row(db1), dw2p, _row(db2p), bn=2048)
    return out[:N, :dw2.shape[1]]


# speedup vs baseline: 2.1892x; 1.0341x over previous
import functools

import jax
import jax.numpy as jnp
from jax import lax
from jax.experimental import pallas as pl
from jax.experimental.pallas import tpu as pltpu
from jax.experimental.pallas import tpu_sc as plsc

N = 10000
E = 320000
H = 128

NC = 2
NS = 16
NW = NC * NS

NP = 10240
EP = 327680
PER_W = EP // NW
C = 128
NCHUNK = PER_W // C
ROWS_PER_SUB = NP // NS

def _sc_mesh():
    return plsc.VectorSubcoreMesh(core_axis_name="c", subcore_axis_name="s",
                                  num_cores=NC, num_subcores=NS)


def _sc_gather2(table, rowp, colp, d):
    kk = 160 if d == H // 2 else 80
    tot = (EP // C) // NS
    kmax = max(kk, tot - kk)

    def body(tab, rid, cid, out_r, out_c, idxs, b00, b01, b10, b11,
             g00, g01, g10, g11, w00, w01, w10, w11):
        cor = lax.axis_index("c")
        sid = lax.axis_index("s")

        bufs = ((b00, b01), (b10, b11))
        gsem = ((g00, g01), (g10, g11))
        wsem = ((w00, w01), (w10, w11))
        outs = (out_r, out_c)

        def run(nch, base):
            if nch == 0:
                return
            pltpu.sync_copy(rid.at[pl.ds(base, nch * C)],
                            idxs.at[0, pl.ds(0, nch * C)])
            pltpu.sync_copy(cid.at[pl.ds(base, nch * C)],
                            idxs.at[1, pl.ds(0, nch * C)])

            def fire(b, p, g):
                pltpu.async_copy(
                    tab.at[idxs.at[b, pl.ds(g * C, C)]], bufs[b][p],
                    gsem[b][p])

            def wait_gather(b, p):
                pltpu.make_async_copy(
                    tab.at[pl.ds(0, C)], bufs[b][p], gsem[b][p]).wait()

            def wait_wb(b, p):
                pltpu.make_async_copy(
                    bufs[b][p], outs[b].at[pl.ds(base, C)], wsem[b][p]).wait()

            for b in (0, 1):
                fire(b, 0, 0)

            @pl.loop(0, nch, step=2)
            def _(g0):
                for p in (0, 1):
                    g = g0 + p
                    for b in (0, 1):
                        wait_gather(b, p)
                        pltpu.async_copy(
                            bufs[b][p], outs[b].at[pl.ds(base + g * C, C)],
                            wsem[b][p])

                        @pl.when(g + 1 < nch)
                        def _():
                            @pl.when(g >= 1)
                            def _():
                                wait_wb(b, 1 - p)
                            fire(b, 1 - p, g + 1)

            for b in (0, 1):
                wait_wb(b, (nch - 1) % 2)

        @pl.when(cor == 0)
        def _():
            run(kk, sid * (kk * C))

        @pl.when(cor == 1)
        def _():
            run(tot - kk, (NS * kk + sid * (tot - kk)) * C)

    fn = pl.kernel(
        body,
        out_type=[
            jax.ShapeDtypeStruct((EP, d), jnp.float32),
            jax.ShapeDtypeStruct((EP, d), jnp.float32),
        ],
        mesh=_sc_mesh(),
        scratch_types=[
            pltpu.VMEM((2, kmax * C), jnp.int32),
            pltpu.VMEM((C, d), jnp.float32),
            pltpu.VMEM((C, d), jnp.float32),
            pltpu.VMEM((C, d), jnp.float32),
            pltpu.VMEM((C, d), jnp.float32),
            pltpu.SemaphoreType.DMA,
            pltpu.SemaphoreType.DMA,
            pltpu.SemaphoreType.DMA,
            pltpu.SemaphoreType.DMA,
            pltpu.SemaphoreType.DMA,
            pltpu.SemaphoreType.DMA,
            pltpu.SemaphoreType.DMA,
            pltpu.SemaphoreType.DMA,
        ],
        name=f"sc_gather2_{d}",
        compiler_params=pltpu.CompilerParams(use_tc_tiling_on_sc=(d % 128 == 0)),
    )
    return fn(table, rowp, colp)


HALF_E = EP // 2
PER_S = HALF_E // NS


def _sc_scatter_add(msga, msgb, colp, zeros_np):
    def body(ma, mb, cidx, zer, out0, out1, idx, buf, acc, sem):
        cid = lax.axis_index("c")
        sid = lax.axis_index("s")
        myrows = pl.ds(sid * ROWS_PER_SUB, ROWS_PER_SUB)
        pltpu.sync_copy(zer.at[myrows], acc.at[myrows])
        plsc.subcore_barrier()

        def run(m, half_base):
            base = half_base + sid * PER_S

            @pl.loop(0, PER_S // C)
            def _(k):
                off = base + k * C
                pltpu.sync_copy(cidx.at[pl.ds(off, C)], idx)
                cp = pltpu.async_copy(m.at[pl.ds(off - half_base, C)], buf,
                                      sem)
                cp.wait()
                pltpu.sync_copy(buf, acc.at[idx], add=True)

        @pl.when(cid == 0)
        def _():
            run(ma, 0)

        @pl.when(cid == 1)
        def _():
            run(mb, HALF_E)

        plsc.subcore_barrier()

        @pl.when(cid == 0)
        def _():
            pltpu.sync_copy(acc.at[myrows], out0.at[myrows])

        @pl.when(cid == 1)
        def _():
            pltpu.sync_copy(acc.at[myrows], out1.at[myrows])

    fn = pl.kernel(
        body,
        out_type=[
            jax.ShapeDtypeStruct((NP, H), jnp.float32),
            jax.ShapeDtypeStruct((NP, H), jnp.float32),
        ],
        mesh=_sc_mesh(),
        scratch_types=[
            pltpu.VMEM((C,), jnp.int32),
            pltpu.VMEM((C, H), jnp.float32),
            pltpu.VMEM_SHARED((NP, H), jnp.float32),
            pltpu.SemaphoreType.DMA,
        ],
        name="sc_scatter_add",
    )
    return fn(msga, msgb, colp, zeros_np)


def _ln(y, g, be):
    mu = jnp.mean(y, axis=-1, keepdims=True)
    var = jnp.mean((y - mu) * (y - mu), axis=-1, keepdims=True)
    return (y - mu) * lax.rsqrt(var + 1e-5) * g + be


def _full(shape):
    return pl.BlockSpec(shape, lambda i: (0,) * len(shape))


def _rows(b, d):
    return pl.BlockSpec((b, d), lambda i: (i, 0))


def _pack_bf16(o):
    u = lax.bitcast_convert_type(o, jnp.uint32)
    r = u + jnp.uint32(0x7FFF) + ((u >> 16) & jnp.uint32(1))
    lo = r[:, :H // 2] >> 16
    hi = r[:, H // 2:] & jnp.uint32(0xFFFF0000)
    return lax.bitcast_convert_type(hi | lo, jnp.float32)


def _unpack_bf16(p):
    u = lax.bitcast_convert_type(p, jnp.uint32)
    lo = lax.bitcast_convert_type(u << 16, jnp.float32)
    hi = lax.bitcast_convert_type(u & jnp.uint32(0xFFFF0000), jnp.float32)
    return jnp.concatenate([lo, hi], axis=1)


def _encode_body(x_ref, w1, b1, w2, b2, g, be, o_ref, op_ref):
    h = jnp.maximum(x_ref[...] @ w1[...] + b1[...], 0.0)
    o = _ln(h @ w2[...] + b2[...], g[...], be[...])
    o_ref[...] = o
    op_ref[...] = _pack_bf16(o)


def _tc_encode(x, w1, b1, w2, b2, g, be, bn):
    n, din = x.shape
    return pl.pallas_call(
        _encode_body,
        grid=(n // bn,),
        in_specs=[
            _rows(bn, din),
            _full(w1.shape), _full(b1.shape), _full(w2.shape),
            _full(b2.shape), _full(g.shape), _full(be.shape),
        ],
        out_specs=[_rows(bn, H), _rows(bn, H // 2)],
        out_shape=[
            jax.ShapeDtypeStruct((n, H), jnp.float32),
            jax.ShapeDtypeStruct((n, H // 2), jnp.float32),
        ],
        name="tc_encode",
    )(x, w1, b1, w2, b2, g, be)


def _edge_enc_body(pr_ref, pc_ref, ssel, wd, wdist, b1, w2, b2, g, be, o_ref):
    d = pr_ref[...] - pc_ref[...]
    s2 = (d * d) @ ssel[...]
    dist = jnp.sqrt(s2 + 1e-12)
    h = jnp.maximum(d @ wd[...] + dist @ wdist[...] + b1[...], 0.0)
    o_ref[...] = _ln(h @ w2[...] + b2[...], g[...], be[...])


def _tc_edge_encode(pr, pc, ssel, wd, wdist, b1, w2, b2, g, be, bn):
    return pl.pallas_call(
        _edge_enc_body,
        grid=(EP // bn,),
        in_specs=[
            _rows(bn, pr.shape[1]), _rows(bn, pc.shape[1]),
            _full(ssel.shape), _full(wd.shape), _full(wdist.shape),
            _full(b1.shape), _full(w2.shape), _full(b2.shape),
            _full(g.shape), _full(be.shape),
        ],
        out_specs=_rows(bn, H),
        out_shape=jax.ShapeDtypeStruct((EP, H), jnp.float32),
        name="tc_edge_encode",
    )(pr, pc, ssel, wd, wdist, b1, w2, b2, g, be)


def _dot(a, b):
    return jnp.dot(a, b, preferred_element_type=jnp.float32)


def _unpack_pair(p):
    u = lax.bitcast_convert_type(p, jnp.uint32)
    lo = lax.bitcast_convert_type(u << 16, jnp.float32)
    hi = lax.bitcast_convert_type(u & jnp.uint32(0xFFFF0000), jnp.float32)
    d2 = H // 2
    xa = jnp.concatenate([lo[:, :d2], hi[:, :d2]], axis=1)
    xb = jnp.concatenate([lo[:, d2:], hi[:, d2:]], axis=1)
    return jnp.concatenate([xa, xb], axis=0)


def _edge_step_body(xr_ref, xc_ref, eha_ref, ehb_ref, w1ab, w1c, b1, w2, b2,
                    g, be, msga_ref, msgb_ref, nea_ref, neb_ref):
    eh = jnp.concatenate([eha_ref[...], ehb_ref[...]], axis=0)
    xr = _unpack_pair(xr_ref[...])
    xc = _unpack_pair(xc_ref[...])
    mc = _dot(xc, w1ab[...])
    mr = _dot(xr, w1ab[...])
    ec = _dot(eh, w1c[...]) + b1[...]
    pre_m = mc[:, :H] + mr[:, H:] + ec
    pre_n = mr[:, :H] + mc[:, H:] + ec
    hm = jnp.maximum(pre_m, 0.0)
    hn = jnp.maximum(pre_n, 0.0)
    msg = _ln(_dot(hm, w2[...]) + b2[...], g[...], be[...])
    ne = eh + _ln(_dot(hn, w2[...]) + b2[...], g[...], be[...])
    bn2 = msga_ref.shape[0]
    msga_ref[...] = msg[:bn2]
    msgb_ref[...] = msg[bn2:]
    nea_ref[...] = ne[:bn2]
    neb_ref[...] = ne[bn2:]


def _tc_edge_step(xr, xc, eha, ehb, eh_split, w1ab, w1c, b1, w2, b2, g, be,
                  bn):
    bn2 = bn // 2
    nblk = HALF_E // bn2
    ehb_map = (lambda i: (i + nblk, 0)) if eh_split else (lambda i: (i, 0))
    half_shape = jax.ShapeDtypeStruct((HALF_E, H), jnp.float32)
    return pl.pallas_call(
        _edge_step_body,
        grid=(nblk,),
        in_specs=[
            _rows(bn2, H), _rows(bn2, H), _rows(bn2, H),
            pl.BlockSpec((bn2, H), ehb_map),
            _full(w1ab.shape), _full(w1c.shape), _full(b1.shape),
            _full(w2.shape), _full(b2.shape), _full(g.shape), _full(be.shape),
        ],
        out_specs=[_rows(bn2, H)] * 4,
        out_shape=[half_shape] * 4,
        name="tc_edge_step",
    )(xr, xc, eha, ehb, w1ab, w1c, b1, w2, b2, g, be)


def _node_step_body(a0_ref, a1_ref, xh_ref, wna, wnb, b1, w2, b2, g, be,
                    o_ref, op_ref):
    xh = xh_ref[...]
    pre = (a0_ref[...] + a1_ref[...]) @ wna[...] + xh @ wnb[...] + b1[...]
    h = jnp.maximum(pre, 0.0)
    o = xh + _ln(h @ w2[...] + b2[...], g[...], be[...])
    o_ref[...] = o
    op_ref[...] = _pack_bf16(o)


def _tc_node_step(a0, a1, xh, wna, wnb, b1, w2, b2, g, be, bn):
    return pl.pallas_call(
        _node_step_body,
        grid=(NP // bn,),
        in_specs=[
            _rows(bn, H), _rows(bn, H), _rows(bn, H),
            _full(wna.shape), _full(wnb.shape), _full(b1.shape),
            _full(w2.shape), _full(b2.shape), _full(g.shape), _full(be.shape),
        ],
        out_specs=[_rows(bn, H), _rows(bn, H // 2)],
        out_shape=[
            jax.ShapeDtypeStruct((NP, H), jnp.float32),
            jax.ShapeDtypeStruct((NP, H // 2), jnp.float32),
        ],
        name="tc_node_step",
    )(a0, a1, xh, wna, wnb, b1, w2, b2, g, be)


def _decode_body(xh_ref, w1, b1, w2, b2, o_ref):
    h = jnp.maximum(xh_ref[...] @ w1[...] + b1[...], 0.0)
    o_ref[...] = h @ w2[...] + b2[...]


def _tc_decode(xh, w1, b1, w2, b2, bn):
    return pl.pallas_call(
        _decode_body,
        grid=(NP // bn,),
        in_specs=[
            _rows(bn, H),
            _full(w1.shape), _full(b1.shape), _full(w2.shape), _full(b2.shape),
        ],
        out_specs=_rows(bn, H),
        out_shape=jax.ShapeDtypeStruct((NP, H), jnp.float32),
        name="tc_decode",
    )(xh, w1, b1, w2, b2)


def _fold_norm(w1, b1, mu, sig):
    w1f = w1 / sig[:, None]
    b1f = b1 - (mu / sig) @ w1
    return w1f, b1f


def _row(v):
    return v.reshape(1, -1)


def kernel(world_pos, mesh_pos, phi, swelling_phi, swelling_phi_rate,
           node_type, time, mat_param, edge_index, params):
    f32 = jnp.float32

    u = world_pos - mesh_pos
    freqs = 2.0 ** jnp.arange(2, dtype=f32)
    temb = jnp.concatenate([jnp.sin(freqs * time), jnp.cos(freqs * time)])
    x = jnp.concatenate(
        [u, phi, swelling_phi, swelling_phi_rate, node_type,
         jnp.tile(temb[None, :], (N, 1)), jnp.tile(mat_param[None, :], (N, 1))],
        axis=-1)
    x = jnp.pad(x, ((0, NP - N), (0, 0)))

    ptab = jnp.concatenate(
        [mesh_pos, world_pos, phi, jnp.zeros((N, 11), f32)], axis=-1)
    ptab = jnp.pad(ptab, ((0, NP - N), (0, 0)))

    rowp = jnp.full((EP,), N, jnp.int32).at[:E].set(edge_index[0])
    colp = jnp.full((EP,), N, jnp.int32).at[:E].set(edge_index[1])
    rowi = jnp.stack([rowp[:EP // 2], rowp[EP // 2:]], axis=1).reshape(EP)
    coli = jnp.stack([colp[:EP // 2], colp[EP // 2:]], axis=1).reshape(EP)

    nmu, nsig = params["node_norm"]
    (wn1, bn1), (wn2, bn2) = params["node_encoder"]["layers"]
    gn, ben = params["node_encoder"]["ln"]
    wn1f, bn1f = _fold_norm(wn1, bn1, nmu, nsig)

    emu, esig = params["edge_norm"]
    (we1, be1), (we2, be2) = params["edge_encoder"]["layers"]
    ge, bee = params["edge_encoder"]["ln"]
    we1f, be1f = _fold_norm(we1, be1, emu, esig)
    wd = jnp.zeros((16, H), f32)
    wd = wd.at[0].set(we1f[0]).at[1].set(we1f[1])
    wd = wd.at[2].set(we1f[3]).at[3].set(we1f[4])
    wd = wd.at[4].set(we1f[6])
    wdist = jnp.stack([we1f[2], we1f[5]], axis=0)
    ssel = jnp.zeros((16, 2), f32)
    ssel = ssel.at[0, 0].set(1.0).at[1, 0].set(1.0)
    ssel = ssel.at[2, 1].set(1.0).at[3, 1].set(1.0)

    x_h, xpk = _tc_encode(x, wn1f, _row(bn1f), wn2, _row(bn2), _row(gn),
                          _row(ben), bn=2048)

    pr, pc = _sc_gather2(ptab, rowp, colp, 16)
    e_h = _tc_edge_encode(pr, pc, ssel, wd, wdist, _row(be1f), we2,
                          _row(be2), _row(ge), _row(bee), bn=2048)
    eha = ehb = e_h
    eh_split = True

    zeros_np = jnp.zeros((NP, H), f32)

    for proc in params["processors"]:
        (pw1, pb1), (pw2, pb2) = proc["edge_mlp"]["layers"]
        pg, pbe = proc["edge_mlp"]["ln"]
        w1a = pw1[0:H]
        w1b = pw1[H:2 * H]
        w1c = pw1[2 * H:3 * H]
        w1ab = jnp.concatenate([w1a, w1b], axis=1)

        (nw1, nb1), (nw2, nb2) = proc["node_mlp"]["layers"]
        ng, nbe = proc["node_mlp"]["ln"]
        wna = nw1[0:H]
        wnb = nw1[H:2 * H]

        xr64, xc64 = _sc_gather2(xpk, rowi, coli, H // 2)
        xrp = xr64.reshape(HALF_E, H)
        xcp = xc64.reshape(HALF_E, H)
        msga, msgb, eha, ehb = _tc_edge_step(
            xrp, xcp, eha, ehb, eh_split, w1ab, w1c, _row(pb1), pw2,
            _row(pb2), _row(pg), _row(pbe), bn=1024)
        eh_split = False
        a0, a1 = _sc_scatter_add(msga, msgb, colp, zeros_np)
        x_h, xpk = _tc_node_step(a0, a1, x_h, wna, wnb, _row(nb1), nw2,
                                 _row(nb2), _row(ng), _row(nbe), bn=2048)

    (dw1, db1), (dw2, db2) = params["node_decoder"]["layers"]
    dw2p = jnp.zeros((H, H), f32).at[:, :dw2.shape[1]].set(dw2)
    db2p = jnp.zeros((H,), f32).at[:dw2.shape[1]].set(db2)
    out = _tc_decode(x_h, dw1, _row(db1), dw2p, _row(db2p), bn=2048)
    return out[:N, :dw2.shape[1]]
